# baseline (speedup 1.0000x reference)
import jax
import jax.numpy as jnp
from jax import lax
from jax.experimental import pallas as pl
from jax.experimental.pallas import tpu as pltpu
from jax.experimental.pallas import tpu_sc as plsc

_B, _L, _F = 16384, 50, 10
_DS, _DT = 32, 32
_NSYM, _NTIME = 100, 2000
_DO = _F - 2 + _DS + _DT
_CT = _DO // 8
_DSP, _DTP = _DS + 1, _DT + 1

_NC, _NS, _LANES = 2, 16, 16
_NW = _NC * _NS
_BT = _B // 128
_NUNIT = _L * _BT
_UPW = _NUNIT // _NW
_G = 128 // _LANES


def _body(xt_hbm, embs_hbm, embt_hbm, out_hbm,
          embs_v, embt_v, xv0, xv1, ov0, ov1, sin, sout, sx, stab):
    wid = lax.axis_index("s") * _NC + lax.axis_index("c")
    base = wid * _UPW
    xvs = (xv0, xv1)
    ovs = (ov0, ov1)

    pltpu.async_copy(embs_hbm, embs_v.at[:, pl.ds(0, _DS)], stab).wait()
    pltpu.async_copy(embt_hbm, embt_v.at[:, pl.ds(0, _DT)], stab).wait()

    iota = lax.iota(jnp.int32, _LANES)

    def start_in(u, b):
        l, bt = u // _BT, u % _BT
        return pltpu.async_copy(
            xt_hbm.at[pl.ds(_F - 2, 2), l, pl.ds(bt * 128, 128)],
            xvs[b], sin[b])

    def wait_in(b):
        pltpu.make_async_copy(xt_hbm.at[pl.ds(_F - 2, 2), 0, pl.ds(0, 128)],
                              xvs[b], sin[b]).wait()

    def start_x(u, b):
        l, bt = u // _BT, u % _BT
        return pltpu.async_copy(
            xt_hbm.at[pl.ds(0, _F - 2), l, pl.ds(bt * 128, 128)],
            ovs[b].at[0], sx[b])

    def wait_x(b):
        pltpu.make_async_copy(xt_hbm.at[pl.ds(0, _F - 2), 0, pl.ds(0, 128)],
                              ovs[b].at[0], sx[b]).wait()

    def start_out(u, b):
        l, bt = u // _BT, u % _BT
        return pltpu.async_copy(ovs[b], out_hbm.at[l, :, bt], sout[b])

    def wait_out(b):
        pltpu.make_async_copy(out_hbm.at[0, :, 0], ovs[b], sout[b]).wait()

    def compute(b):
        xv, ov = xvs[b], ovs[b]
        srows = [xv[0, pl.ds(g * _LANES, _LANES)].astype(jnp.int32)
                 for g in range(_G)]
        trows = [xv[1, pl.ds(g * _LANES, _LANES)].astype(jnp.int32)
                 for g in range(_G)]
        sr, tr = tuple(srows), tuple(trows)

        @plsc.parallel_loop(0, _LANES, step=1, unroll=2)
        def kstep(k):
            rot = (iota + k) & (_LANES - 1)
            rot16 = rot + _LANES
            for half, dvec in ((0, rot), (1, rot16)):
                cs = (_F - 2) + half * _LANES + rot
                cts, cis = cs >> 3, cs & 7
                cv = (_F - 2 + _DS) + half * _LANES + rot
                ctt, cit = cv >> 3, cv & 7
                vals = []
                for g in range(_G):
                    vals.append(plsc.load_gather(embs_v, [sr[g], dvec]))
                    vals.append(plsc.load_gather(embt_v, [tr[g], dvec]))
                for g in range(_G):
                    col = iota + g * _LANES
                    plsc.store_scatter(ov, [cts, cis, col], vals[2 * g])
                    plsc.store_scatter(ov, [ctt, cit, col], vals[2 * g + 1])

    start_in(base, 0)
    start_in(base + 1, 1)

    def pair(j, carry):
        u0 = base + j * 2
        for b in range(2):
            u = u0 + b
            @pl.when(j * 2 + b >= 2)
            def _():
                wait_out(b)
            start_x(u, b)
            wait_in(b)
            compute(b)
            wait_x(b)
            start_out(u, b)

            @pl.when(j * 2 + b + 2 < _UPW)
            def _():
                start_in(u + 2, b)
        return carry

    lax.fori_loop(0, _UPW // 2, pair, 0)
    wait_out(0)
    wait_out(1)


@jax.jit
def _sc_embed(xt, embs, embt):
    mesh = plsc.VectorSubcoreMesh(core_axis_name="c", subcore_axis_name="s")
    return pl.kernel(
        _body,
        out_type=jax.ShapeDtypeStruct((_L, _CT, _BT, 8, 128), jnp.float32),
        mesh=mesh,
        compiler_params=pltpu.CompilerParams(needs_layout_passes=False,
                                             use_tc_tiling_on_sc=False),
        scratch_types=[
            pltpu.VMEM((_NSYM, _DSP), jnp.float32),
            pltpu.VMEM((_NTIME, _DTP), jnp.float32),
            pltpu.VMEM((2, 128), jnp.float32),
            pltpu.VMEM((2, 128), jnp.float32),
            pltpu.VMEM((_CT, 8, 128), jnp.float32),
            pltpu.VMEM((_CT, 8, 128), jnp.float32),
            [pltpu.SemaphoreType.DMA, pltpu.SemaphoreType.DMA],
            [pltpu.SemaphoreType.DMA, pltpu.SemaphoreType.DMA],
            [pltpu.SemaphoreType.DMA, pltpu.SemaphoreType.DMA],
            pltpu.SemaphoreType.DMA,
        ],
    )(xt, embs, embt)


def kernel(x, emb_s, emb_t):
    xt = jnp.transpose(x, (2, 1, 0))
    out5 = _sc_embed(xt, emb_s, emb_t)
    return jnp.transpose(out5, (2, 4, 0, 1, 3)).reshape(_B, _L, _DO)


# speedup vs baseline: 1.0077x; 1.0077x over previous
import jax
import jax.numpy as jnp
from jax import lax
from jax.experimental import pallas as pl
from jax.experimental.pallas import tpu as pltpu
from jax.experimental.pallas import tpu_sc as plsc

_B, _L, _F = 16384, 50, 10
_DS, _DT = 32, 32
_NSYM, _NTIME = 100, 2000
_DO = _F - 2 + _DS + _DT
_CT = _DO // 8
_DSP, _DTP = _DS + 1, _DT + 1

_NC, _NS, _LANES = 2, 16, 16
_NW = _NC * _NS
_BT = _B // 128
_NUNIT = _L * _BT
_UPW = _NUNIT // _NW
_G = 128 // _LANES


def _body(xt_hbm, embs_hbm, embt_hbm, out_hbm,
          embs_v, embt_v, xv0, xv1, ov0, ov1, sin, sout, stab):
    wid = lax.axis_index("s") * _NC + lax.axis_index("c")
    base = wid * _UPW
    xvs = (xv0, xv1)
    ovs = (ov0, ov1)

    pltpu.async_copy(embs_hbm, embs_v.at[:, pl.ds(0, _DS)], stab).wait()
    pltpu.async_copy(embt_hbm, embt_v.at[:, pl.ds(0, _DT)], stab).wait()

    iota = lax.iota(jnp.int32, _LANES)

    def start_in(u, b):
        l, bt = u // _BT, u % _BT
        return pltpu.async_copy(xt_hbm.at[:, l, pl.ds(bt * 128, 128)],
                                xvs[b], sin[b])

    def wait_in(b):
        pltpu.make_async_copy(xt_hbm.at[:, 0, pl.ds(0, 128)], xvs[b],
                              sin[b]).wait()

    def start_out(u, b):
        l, bt = u // _BT, u % _BT
        return pltpu.async_copy(ovs[b], out_hbm.at[l, :, bt], sout[b])

    def wait_out(b):
        pltpu.make_async_copy(out_hbm.at[0, :, 0], ovs[b], sout[b]).wait()

    def compute(b):
        xv, ov = xvs[b], ovs[b]
        for g in range(_G):
            j0 = g * _LANES
            for c in range(_F - 2):
                ov[0, pl.ds(c * 128 + j0, _LANES)] = xv[c, pl.ds(j0, _LANES)]
        srows = [xv[_F - 2, pl.ds(g * _LANES, _LANES)].astype(jnp.int32)
                 for g in range(_G)]
        trows = [xv[_F - 1, pl.ds(g * _LANES, _LANES)].astype(jnp.int32)
                 for g in range(_G)]
        sr, tr = tuple(srows), tuple(trows)

        @plsc.parallel_loop(0, _LANES, step=1, unroll=2)
        def kstep(k):
            rot = (iota + k) & (_LANES - 1)
            rot16 = rot + _LANES
            for half, dvec in ((0, rot), (1, rot16)):
                cs = (_F - 2) + half * _LANES + rot
                cts, els = cs >> 3, (cs & 7) << 7
                cv = (_F - 2 + _DS) + half * _LANES + rot
                ctt, elt = cv >> 3, (cv & 7) << 7
                vals = []
                for g in range(_G):
                    vals.append(plsc.load_gather(embs_v, [sr[g], dvec]))
                    vals.append(plsc.load_gather(embt_v, [tr[g], dvec]))
                for g in range(_G):
                    col = iota + g * _LANES
                    plsc.store_scatter(ov, [cts, els + col], vals[2 * g])
                    plsc.store_scatter(ov, [ctt, elt + col], vals[2 * g + 1])

    start_in(base, 0)
    start_in(base + 1, 1)

    def pair(j, carry):
        u0 = base + j * 2
        for b in range(2):
            u = u0 + b
            @pl.when(j * 2 + b >= 2)
            def _():
                wait_out(b)
            wait_in(b)
            compute(b)
            start_out(u, b)

            @pl.when(j * 2 + b + 2 < _UPW)
            def _():
                start_in(u + 2, b)
        return carry

    lax.fori_loop(0, _UPW // 2, pair, 0)
    wait_out(0)
    wait_out(1)


@jax.jit
def _sc_embed(xt, embs, embt):
    mesh = plsc.VectorSubcoreMesh(core_axis_name="c", subcore_axis_name="s")
    return pl.kernel(
        _body,
        out_type=jax.ShapeDtypeStruct((_L, _CT, _BT, 1024), jnp.float32),
        mesh=mesh,
        compiler_params=pltpu.CompilerParams(needs_layout_passes=False,
                                             use_tc_tiling_on_sc=False),
        scratch_types=[
            pltpu.VMEM((_NSYM, _DSP), jnp.float32),
            pltpu.VMEM((_NTIME, _DTP), jnp.float32),
            pltpu.VMEM((_F, 128), jnp.float32),
            pltpu.VMEM((_F, 128), jnp.float32),
            pltpu.VMEM((_CT, 1024), jnp.float32),
            pltpu.VMEM((_CT, 1024), jnp.float32),
            [pltpu.SemaphoreType.DMA, pltpu.SemaphoreType.DMA],
            [pltpu.SemaphoreType.DMA, pltpu.SemaphoreType.DMA],
            pltpu.SemaphoreType.DMA,
        ],
    )(xt, embs, embt)


def kernel(x, emb_s, emb_t):
    xt = jnp.transpose(x, (2, 1, 0))
    out5 = _sc_embed(xt, emb_s, emb_t).reshape(_L, _CT, _BT, 8, 128)
    return jnp.transpose(out5, (2, 4, 0, 1, 3)).reshape(_B, _L, _DO)


# speedup vs baseline: 1.0237x; 1.0159x over previous
import jax
import jax.numpy as jnp
from jax import lax
from jax.experimental import pallas as pl
from jax.experimental.pallas import tpu as pltpu
from jax.experimental.pallas import tpu_sc as plsc

_B, _L, _F = 16384, 50, 10
_DS, _DT = 32, 32
_NSYM, _NTIME = 100, 2000
_DO = _F - 2 + _DS + _DT
_CT = _DO // 8
_DSP, _DTP = _DS + 1, _DT + 1

_NC, _NS, _LANES = 2, 16, 16
_NW = _NC * _NS
_BT = _B // 128
_NUNIT = _L * _BT
_UPW = _NUNIT // _NW
_G = 128 // _LANES


def _body(xt_hbm, embs_hbm, embt_hbm, out_hbm,
          embs_v, embt_v, xv0, xv1, ov0, ov1, sin, sout, stab):
    wid = lax.axis_index("s") * _NC + lax.axis_index("c")
    base = wid * _UPW
    xvs = (xv0, xv1)
    ovs = (ov0, ov1)

    pltpu.async_copy(embs_hbm, embs_v.at[:, pl.ds(0, _DS)], stab).wait()
    pltpu.async_copy(embt_hbm, embt_v.at[:, pl.ds(0, _DT)], stab).wait()

    iota = lax.iota(jnp.int32, _LANES)

    def start_in(u, b):
        l, bt = u // _BT, u % _BT
        return pltpu.async_copy(xt_hbm.at[:, l, pl.ds(bt * 128, 128)],
                                xvs[b], sin[b])

    def wait_in(b):
        pltpu.make_async_copy(xt_hbm.at[:, 0, pl.ds(0, 128)], xvs[b],
                              sin[b]).wait()

    def start_out(u, b):
        l, bt = u // _BT, u % _BT
        return pltpu.async_copy(ovs[b], out_hbm.at[l, :, bt], sout[b])

    def wait_out(b):
        pltpu.make_async_copy(out_hbm.at[0, :, 0], ovs[b], sout[b]).wait()

    def compute(b):
        xv, ov = xvs[b], ovs[b]
        for g in range(_G):
            j0 = g * _LANES
            for c in range(_F - 2):
                ov[0, pl.ds(c * 128 + j0, _LANES)] = xv[c, pl.ds(j0, _LANES)]
        srows = [xv[_F - 2, pl.ds(g * _LANES, _LANES)].astype(jnp.int32)
                 for g in range(_G)]
        trows = [xv[_F - 1, pl.ds(g * _LANES, _LANES)].astype(jnp.int32)
                 for g in range(_G)]
        sr, tr = tuple(srows), tuple(trows)

        @plsc.parallel_loop(0, _LANES, step=1, unroll=3)
        def kstep(k):
            rot = (iota + k) & (_LANES - 1)
            rot16 = rot + _LANES
            for half, dvec in ((0, rot), (1, rot16)):
                cs = (_F - 2) + half * _LANES + rot
                cts, els = cs >> 3, (cs & 7) << 7
                cv = (_F - 2 + _DS) + half * _LANES + rot
                ctt, elt = cv >> 3, (cv & 7) << 7
                vals = []
                for g in range(_G):
                    vals.append(plsc.load_gather(embs_v, [sr[g], dvec]))
                    vals.append(plsc.load_gather(embt_v, [tr[g], dvec]))
                for g in range(_G):
                    col = iota + g * _LANES
                    plsc.store_scatter(ov, [cts, els + col], vals[2 * g])
                    plsc.store_scatter(ov, [ctt, elt + col], vals[2 * g + 1])

    start_in(base, 0)
    start_in(base + 1, 1)

    def pair(j, carry):
        u0 = base + j * 2
        for b in range(2):
            u = u0 + b
            @pl.when(j * 2 + b >= 2)
            def _():
                wait_out(b)
            wait_in(b)
            compute(b)
            start_out(u, b)

            @pl.when(j * 2 + b + 2 < _UPW)
            def _():
                start_in(u + 2, b)
        return carry

    lax.fori_loop(0, _UPW // 2, pair, 0)
    wait_out(0)
    wait_out(1)


@jax.jit
def _sc_embed(xt, embs, embt):
    mesh = plsc.VectorSubcoreMesh(core_axis_name="c", subcore_axis_name="s")
    return pl.kernel(
        _body,
        out_type=jax.ShapeDtypeStruct((_L, _CT, _BT, 1024), jnp.float32),
        mesh=mesh,
        compiler_params=pltpu.CompilerParams(needs_layout_passes=False,
                                             use_tc_tiling_on_sc=False),
        scratch_types=[
            pltpu.VMEM((_NSYM, _DSP), jnp.float32),
            pltpu.VMEM((_NTIME, _DTP), jnp.float32),
            pltpu.VMEM((_F, 128), jnp.float32),
            pltpu.VMEM((_F, 128), jnp.float32),
            pltpu.VMEM((_CT, 1024), jnp.float32),
            pltpu.VMEM((_CT, 1024), jnp.float32),
            [pltpu.SemaphoreType.DMA, pltpu.SemaphoreType.DMA],
            [pltpu.SemaphoreType.DMA, pltpu.SemaphoreType.DMA],
            pltpu.SemaphoreType.DMA,
        ],
    )(xt, embs, embt)


def kernel(x, emb_s, emb_t):
    xt = jnp.transpose(x, (2, 1, 0))
    out5 = _sc_embed(xt, emb_s, emb_t).reshape(_L, _CT, _BT, 8, 128)
    return jnp.transpose(out5, (2, 4, 0, 1, 3)).reshape(_B, _L, _DO)


# speedup vs baseline: 1.0306x; 1.0067x over previous
import jax
import jax.numpy as jnp
from jax import lax
from jax.experimental import pallas as pl
from jax.experimental.pallas import tpu as pltpu
from jax.experimental.pallas import tpu_sc as plsc

_B, _L, _F = 16384, 50, 10
_DS, _DT = 32, 32
_NSYM, _NTIME = 100, 2000
_DO = _F - 2 + _DS + _DT
_CT = _DO // 8
_DSP, _DTP = _DS + 1, _DT + 1

_NC, _NS, _LANES = 2, 16, 16
_NW = _NC * _NS
_BT = _B // 128
_NUNIT = _L * _BT
_UPW = _NUNIT // _NW
_G = 128 // _LANES


def _body(xt_hbm, embs_hbm, embt_hbm, out_hbm,
          embs_v, embt_v, xv0, xv1, ov0, ov1, sin, sout, stab):
    wid = lax.axis_index("s") * _NC + lax.axis_index("c")
    base = wid * _UPW
    xvs = (xv0, xv1)
    ovs = (ov0, ov1)

    pltpu.async_copy(embs_hbm, embs_v.at[:, pl.ds(0, _DS)], stab).wait()
    pltpu.async_copy(embt_hbm, embt_v.at[:, pl.ds(0, _DT)], stab).wait()

    iota = lax.iota(jnp.int32, _LANES)

    def start_in(u, b):
        l, bt = u // _BT, u % _BT
        return pltpu.async_copy(xt_hbm.at[:, l, pl.ds(bt * 128, 128)],
                                xvs[b], sin[b])

    def wait_in(b):
        pltpu.make_async_copy(xt_hbm.at[:, 0, pl.ds(0, 128)], xvs[b],
                              sin[b]).wait()

    def start_out(u, b):
        l, bt = u // _BT, u % _BT
        return pltpu.async_copy(ovs[b], out_hbm.at[l, :, bt], sout[b])

    def wait_out(b):
        pltpu.make_async_copy(out_hbm.at[0, :, 0], ovs[b], sout[b]).wait()

    def compute(b):
        xv, ov = xvs[b], ovs[b]
        for g in range(_G):
            j0 = g * _LANES
            for c in range(_F - 2):
                ov[0, pl.ds(c * 128 + j0, _LANES)] = xv[c, pl.ds(j0, _LANES)]
        srows = [xv[_F - 2, pl.ds(g * _LANES, _LANES)].astype(jnp.int32)
                 for g in range(_G)]
        trows = [xv[_F - 1, pl.ds(g * _LANES, _LANES)].astype(jnp.int32)
                 for g in range(_G)]
        sr, tr = tuple(srows), tuple(trows)

        @plsc.parallel_loop(0, _LANES, step=1, unroll=2)
        def kstep(k):
            rot = (iota + k) & (_LANES - 1)
            rot16 = rot + _LANES
            for half, dvec in ((0, rot), (1, rot16)):
                cs = (_F - 2) + half * _LANES + rot
                cts, els = cs >> 3, (cs & 7) << 7
                cv = (_F - 2 + _DS) + half * _LANES + rot
                ctt, elt = cv >> 3, (cv & 7) << 7
                vals = []
                for g in range(_G):
                    vals.append(plsc.load_gather(embs_v, [sr[g], dvec]))
                    vals.append(plsc.load_gather(embt_v, [tr[g], dvec]))
                for g in range(_G):
                    col = iota + g * _LANES
                    plsc.store_scatter(ov, [cts, els + col], vals[2 * g])
                    plsc.store_scatter(ov, [ctt, elt + col], vals[2 * g + 1])

    start_in(base, 0)
    start_in(base + 1, 1)

    def pair(j, carry):
        u0 = base + j * 2
        for b in range(2):
            u = u0 + b
            @pl.when(j * 2 + b >= 2)
            def _():
                wait_out(b)
            wait_in(b)
            compute(b)
            start_out(u, b)

            @pl.when(j * 2 + b + 2 < _UPW)
            def _():
                start_in(u + 2, b)
        return carry

    lax.fori_loop(0, _UPW // 2, pair, 0)
    wait_out(0)
    wait_out(1)


@jax.jit
def _sc_embed(xt, embs, embt):
    mesh = plsc.VectorSubcoreMesh(core_axis_name="c", subcore_axis_name="s")
    return pl.kernel(
        _body,
        out_type=jax.ShapeDtypeStruct((_L, _CT, _BT, 1024), jnp.float32),
        mesh=mesh,
        compiler_params=pltpu.CompilerParams(needs_layout_passes=False,
                                             use_tc_tiling_on_sc=False),
        scratch_types=[
            pltpu.VMEM((_NSYM, _DSP), jnp.float32),
            pltpu.VMEM((_NTIME, _DTP), jnp.float32),
            pltpu.VMEM((_F, 128), jnp.float32),
            pltpu.VMEM((_F, 128), jnp.float32),
            pltpu.VMEM((_CT, 1024), jnp.float32),
            pltpu.VMEM((_CT, 1024), jnp.float32),
            [pltpu.SemaphoreType.DMA, pltpu.SemaphoreType.DMA],
            [pltpu.SemaphoreType.DMA, pltpu.SemaphoreType.DMA],
            pltpu.SemaphoreType.DMA,
        ],
    )(xt, embs, embt)


def kernel(x, emb_s, emb_t):
    xt = jnp.transpose(x, (2, 1, 0))
    out5 = _sc_embed(xt, emb_s, emb_t).reshape(_L, _CT, _BT, 8, 128)
    return jnp.transpose(out5, (2, 4, 0, 1, 3)).reshape(_B, _L, _DO)
